# baseline (speedup 1.0000x reference)
import functools

import jax
import jax.numpy as jnp
from jax import lax
from jax.experimental import pallas as pl
from jax.experimental.pallas import tpu as pltpu
from jax.experimental.pallas import tpu_sc as plsc

NC = 2
NS = 16
NW = NC * NS

VOCAB = 1_000_000
EMBED_DIM = 32
BATCH = 16384
HIST = 200
B_TOTAL = BATCH * HIST
ROWS_PER_W = BATCH // NW


def _gather_body(table_hbm, idx_hbm, out_hbm, idx_v, rows_v, sem):
    wid = lax.axis_index("s") * NC + lax.axis_index("c")
    wbase = wid * ROWS_PER_W

    @pl.loop(0, ROWS_PER_W)
    def _chunk(g):
        b = wbase + g
        pltpu.sync_copy(idx_hbm.at[pl.ds(b * HIST, HIST)], idx_v)
        pltpu.async_copy(table_hbm.at[idx_v], rows_v, sem).wait()
        pltpu.sync_copy(rows_v, out_hbm.at[b])


_gather = functools.partial(
    pl.kernel,
    out_type=jax.ShapeDtypeStruct((BATCH, HIST, EMBED_DIM), jnp.float32),
    mesh=plsc.VectorSubcoreMesh(
        core_axis_name="c", subcore_axis_name="s", num_cores=NC, num_subcores=NS
    ),
    scratch_types=[
        pltpu.VMEM((HIST,), jnp.int32),
        pltpu.VMEM((HIST, EMBED_DIM), jnp.float32),
        pltpu.SemaphoreType.DMA,
    ],
    compiler_params=pltpu.CompilerParams(use_tc_tiling_on_sc=False),
)(_gather_body)


@jax.jit
def kernel(input_ids, table):
    ids = input_ids.reshape(-1).astype(jnp.int32)
    return _gather(table, ids)


# speedup vs baseline: 1.2588x; 1.2588x over previous
import functools

import jax
import jax.numpy as jnp
from jax import lax
from jax.experimental import pallas as pl
from jax.experimental.pallas import tpu as pltpu
from jax.experimental.pallas import tpu_sc as plsc

NC = 2
NS = 16
NW = NC * NS

VOCAB = 1_000_000
EMBED_DIM = 32
BATCH = 16384
HIST = 200
B_TOTAL = BATCH * HIST
B_PER_W = B_TOTAL // NW
CHUNK = 1024
N_CHUNKS = B_PER_W // CHUNK


def _gather_body(table_hbm, idx_hbm, out_hbm, idx_v, rows_v, *sems):
    sem_i = sems[0:2]
    sem_g = sems[2:4]
    sem_w = sems[4:6]
    wid = lax.axis_index("s") * NC + lax.axis_index("c")
    wbase = wid * B_PER_W

    def idx_src(g):
        return idx_hbm.at[pl.ds(wbase + g * CHUNK, CHUNK)]

    def out_dst(g):
        return out_hbm.at[pl.ds(wbase + g * CHUNK, CHUNK)]

    def step(g, k, *, first, last, prefetch=True):
        if not first:
            pltpu.make_async_copy(rows_v.at[k], out_dst(g - 2), sem_w[k]).wait()
        pltpu.async_copy(table_hbm.at[idx_v.at[k]], rows_v.at[k], sem_g[k]).wait()
        if prefetch:
            pltpu.async_copy(idx_src(g + 2), idx_v.at[k], sem_i[k])
        pltpu.async_copy(rows_v.at[k], out_dst(g), sem_w[k])
        if not last:
            pltpu.make_async_copy(idx_src(g + 1), idx_v.at[1 - k], sem_i[1 - k]).wait()

    pltpu.async_copy(idx_src(0), idx_v.at[0], sem_i[0])
    pltpu.make_async_copy(idx_src(0), idx_v.at[0], sem_i[0]).wait()
    pltpu.async_copy(idx_src(1), idx_v.at[1], sem_i[1])
    step(0, 0, first=True, last=False)
    step(1, 1, first=True, last=False)

    @pl.loop(0, (N_CHUNKS - 4) // 2)
    def _pair(h):
        g = 2 + h * 2
        step(g, 0, first=False, last=False)
        step(g + 1, 1, first=False, last=False)

    step(N_CHUNKS - 2, 0, first=False, last=False, prefetch=False)
    step(N_CHUNKS - 1, 1, first=False, last=True, prefetch=False)

    pltpu.make_async_copy(rows_v.at[0], out_dst(N_CHUNKS - 2), sem_w[0]).wait()
    pltpu.make_async_copy(rows_v.at[1], out_dst(N_CHUNKS - 1), sem_w[1]).wait()


_gather = functools.partial(
    pl.kernel,
    out_type=jax.ShapeDtypeStruct((B_TOTAL, EMBED_DIM), jnp.float32),
    mesh=plsc.VectorSubcoreMesh(
        core_axis_name="c", subcore_axis_name="s", num_cores=NC, num_subcores=NS
    ),
    scratch_types=[
        pltpu.VMEM((2, CHUNK), jnp.int32),
        pltpu.VMEM((2, CHUNK, EMBED_DIM), jnp.float32),
        pltpu.SemaphoreType.DMA,
        pltpu.SemaphoreType.DMA,
        pltpu.SemaphoreType.DMA,
        pltpu.SemaphoreType.DMA,
        pltpu.SemaphoreType.DMA,
        pltpu.SemaphoreType.DMA,
    ],
    compiler_params=pltpu.CompilerParams(use_tc_tiling_on_sc=False),
)(_gather_body)


@jax.jit
def kernel(input_ids, table):
    ids = input_ids.reshape(-1).astype(jnp.int32)
    out = _gather(table, ids)
    return out.reshape(*input_ids.shape, table.shape[1])
